# Initial kernel scaffold; baseline (speedup 1.0000x reference)
import functools

import jax
import jax.numpy as jnp
from jax import lax
from jax.experimental import pallas as pl
from jax.experimental.pallas import tpu as pltpu
from jax.experimental.pallas import tpu_sc as plsc

N_NODES = 10000
E = 320000
D = 128
EPS = 1e-5

NUM_CORES = 2
NUM_SUBCORES = 16
DEG_W = 16

EDGES_PER_TILE = E // NUM_SUBCORES
CHUNK = 80
CHUNKS_PER_TILE = EDGES_PER_TILE // CHUNK
ROWS_PER_TILE = N_NODES // NUM_SUBCORES


def _sc_segment_sums(win_feat, win_dst, loss_feat, loss_dst):
    zeros_h = jnp.zeros((N_NODES, D), dtype=jnp.float32)
    ones_h = jnp.ones((CHUNK, DEG_W), dtype=jnp.float32)

    mesh = plsc.VectorSubcoreMesh(
        core_axis_name="c", subcore_axis_name="s",
        num_cores=NUM_CORES, num_subcores=NUM_SUBCORES)

    @functools.partial(
        pl.kernel,
        out_type=[
            jax.ShapeDtypeStruct((N_NODES, D), jnp.float32),
            jax.ShapeDtypeStruct((N_NODES, DEG_W), jnp.float32),
            jax.ShapeDtypeStruct((N_NODES, D), jnp.float32),
            jax.ShapeDtypeStruct((N_NODES, DEG_W), jnp.float32),
        ],
        mesh=mesh,
        scratch_types=[
            pltpu.VMEM_SHARED((N_NODES, D), jnp.float32),
            pltpu.VMEM_SHARED((N_NODES, DEG_W), jnp.float32),
            pltpu.VMEM((CHUNK,), jnp.int32),
            pltpu.VMEM((CHUNK, D), jnp.float32),
            pltpu.VMEM((CHUNK, DEG_W), jnp.float32),
        ],
    )
    def sc_kernel(wf_hbm, wd_hbm, lf_hbm, ld_hbm, z_hbm, o_hbm,
                  sum_w_hbm, deg_w_hbm, sum_l_hbm, deg_l_hbm,
                  acc, deg, idx_v, feat_v, ones_v):
        c = lax.axis_index("c")
        s = lax.axis_index("s")
        row0 = s * ROWS_PER_TILE
        base_t = s * EDGES_PER_TILE

        def run(feat_hbm, dst_hbm, sum_out, deg_out):
            pltpu.sync_copy(z_hbm.at[pl.ds(row0, ROWS_PER_TILE), :],
                            acc.at[pl.ds(row0, ROWS_PER_TILE), :])
            pltpu.sync_copy(z_hbm.at[pl.ds(row0, ROWS_PER_TILE), pl.ds(0, DEG_W)],
                            deg.at[pl.ds(row0, ROWS_PER_TILE), :])
            pltpu.sync_copy(o_hbm, ones_v)
            plsc.subcore_barrier()

            def body(i, carry):
                base = base_t + i * CHUNK
                pltpu.sync_copy(dst_hbm.at[pl.ds(base, CHUNK)], idx_v)
                pltpu.sync_copy(feat_hbm.at[pl.ds(base, CHUNK), :], feat_v)
                pltpu.sync_copy(feat_v, acc.at[idx_v], add=True)
                pltpu.sync_copy(ones_v, deg.at[idx_v], add=True)
                return carry

            lax.fori_loop(0, CHUNKS_PER_TILE, body, 0)
            plsc.subcore_barrier()

            pltpu.sync_copy(acc.at[pl.ds(row0, ROWS_PER_TILE), :],
                            sum_out.at[pl.ds(row0, ROWS_PER_TILE), :])
            pltpu.sync_copy(deg.at[pl.ds(row0, ROWS_PER_TILE), :],
                            deg_out.at[pl.ds(row0, ROWS_PER_TILE), :])

        pl.when(c == 0)(lambda: run(wf_hbm, wd_hbm, sum_w_hbm, deg_w_hbm))
        pl.when(c == 1)(lambda: run(lf_hbm, ld_hbm, sum_l_hbm, deg_l_hbm))

    return sc_kernel(win_feat, win_dst, loss_feat, loss_dst, zeros_h, ones_h)


BN_ROWS = 2000


def _readout_body(sw_ref, sl_ref, dw_ref, dl_ref,
                  g_ref, bt_ref, rm_ref, rv_ref, w_ref, b_ref, o_ref):
    dw = dw_ref[:, 0:1]
    dl = dl_ref[:, 0:1]
    mw = sw_ref[...] / jnp.maximum(dw, 1.0)
    ml = sl_ref[...] / jnp.maximum(dl, 1.0)
    hw = (dw > 0.0).astype(jnp.float32)
    hl = (dl > 0.0).astype(jnp.float32)
    cnt = jnp.maximum(hw + hl, 1.0)
    h = (mw * hw + ml * hl) / cnt
    hb = (h - rm_ref[...]) * lax.rsqrt(rv_ref[...] + EPS) * g_ref[...] + bt_ref[...]
    y = jnp.dot(hb, w_ref[...], preferred_element_type=jnp.float32) + b_ref[...]
    o_ref[...] = jnp.maximum(y, 0.0)


def _readout(sum_w, deg_w, sum_l, deg_l, r_gamma, r_beta, r_rm, r_rv, W3, b3):
    grid = (N_NODES // BN_ROWS,)
    row_spec = pl.BlockSpec((BN_ROWS, D), lambda i: (i, 0))
    deg_spec = pl.BlockSpec((BN_ROWS, DEG_W), lambda i: (i, 0))
    vec_spec = pl.BlockSpec((1, D), lambda i: (0, 0))
    mat_spec = pl.BlockSpec((D, D), lambda i: (0, 0))
    return pl.pallas_call(
        _readout_body,
        grid=grid,
        in_specs=[row_spec, row_spec, deg_spec, deg_spec,
                  vec_spec, vec_spec, vec_spec, vec_spec, mat_spec, vec_spec],
        out_specs=row_spec,
        out_shape=jax.ShapeDtypeStruct((N_NODES, D), jnp.float32),
    )(sum_w, sum_l, deg_w, deg_l,
      r_gamma.reshape(1, D), r_beta.reshape(1, D),
      r_rm.reshape(1, D), r_rv.reshape(1, D), W3, b3.reshape(1, D))


def kernel(win_feat, loss_feat, win_dst, loss_dst,
           r_gamma, r_beta, r_rm, r_rv, W3, b3):
    sum_w, deg_w, sum_l, deg_l = _sc_segment_sums(
        win_feat, win_dst, loss_feat, loss_dst)
    return _readout(sum_w, deg_w, sum_l, deg_l,
                    r_gamma, r_beta, r_rm, r_rv, W3, b3)


# speedup vs baseline: 4.1282x; 4.1282x over previous
import functools

import jax
import jax.numpy as jnp
from jax import lax
from jax.experimental import pallas as pl
from jax.experimental.pallas import tpu as pltpu
from jax.experimental.pallas import tpu_sc as plsc

N_NODES = 10000
E = 320000
D = 128
EPS = 1e-5

NUM_CORES = 2
NUM_SUBCORES = 16

EDGES_PER_TILE = E // NUM_SUBCORES
CHUNK = 80
CHUNKS_PER_TILE = EDGES_PER_TILE // CHUNK
ROW_STRIDE = 624
ROW_WIN = 640

def _sc_segment_sums(win_feat, win_dst, loss_feat, loss_dst):
    zeros_h = jnp.zeros((N_NODES, D), dtype=jnp.float32)

    mesh = plsc.VectorSubcoreMesh(
        core_axis_name="c", subcore_axis_name="s",
        num_cores=NUM_CORES, num_subcores=NUM_SUBCORES)

    @functools.partial(
        pl.kernel,
        out_type=[
            jax.ShapeDtypeStruct((N_NODES, D), jnp.float32),
            jax.ShapeDtypeStruct((N_NODES, D), jnp.float32),
            jax.ShapeDtypeStruct((N_NODES,), jnp.float32),
            jax.ShapeDtypeStruct((N_NODES,), jnp.float32),
        ],
        mesh=mesh,
        scratch_types=[
            pltpu.VMEM_SHARED((N_NODES, D), jnp.float32),
            pltpu.VMEM_SHARED((N_NODES,), jnp.float32),
            pltpu.VMEM((CHUNK,), jnp.int32),
            pltpu.VMEM((CHUNK, D), jnp.float32),
            pltpu.VMEM((ROW_WIN,), jnp.float32),
            pltpu.VMEM((CHUNK,), jnp.float32),
        ],
    )
    def sc_kernel(wf_hbm, wd_hbm, lf_hbm, ld_hbm, z_hbm,
                  sum_w_hbm, sum_l_hbm, deg_w_hbm, deg_l_hbm,
                  acc, deg, idx_v, feat_v, dstage_v, ones_v):
        c = lax.axis_index("c")
        s = lax.axis_index("s")
        row0 = s * ROW_STRIDE
        base_t = s * EDGES_PER_TILE

        def _fill(i, val, ref):
            ref[pl.ds(i * 16, 16)] = jnp.full((16,), val, jnp.float32)
            return val

        lax.fori_loop(0, ROW_WIN // 16,
                      lambda i, v: _fill(i, v, dstage_v), 0.0)
        lax.fori_loop(0, CHUNK // 16,
                      lambda i, v: _fill(i, v, ones_v), 1.0)

        for j in range(ROW_WIN // CHUNK):
            r = row0 + j * CHUNK
            pltpu.sync_copy(z_hbm.at[pl.ds(r, CHUNK), :], feat_v)
            pltpu.sync_copy(feat_v, acc.at[pl.ds(r, CHUNK), :])
        pltpu.sync_copy(dstage_v, deg.at[pl.ds(row0, ROW_WIN)])
        plsc.subcore_barrier()

        def scatter(feat_hbm, dst_hbm):
            def body(i, carry):
                base = base_t + i * CHUNK
                pltpu.sync_copy(dst_hbm.at[pl.ds(base, CHUNK)], idx_v)
                pltpu.sync_copy(feat_hbm.at[pl.ds(base, CHUNK), :], feat_v)
                pltpu.sync_copy(feat_v, acc.at[idx_v], add=True)
                pltpu.sync_copy(ones_v, deg.at[idx_v], add=True)
                return carry

            lax.fori_loop(0, CHUNKS_PER_TILE, body, 0)

        pl.when(c == 0)(lambda: scatter(wf_hbm, wd_hbm))
        pl.when(c == 1)(lambda: scatter(lf_hbm, ld_hbm))
        plsc.subcore_barrier()

        def writeback(sum_out, deg_out):
            for j in range(ROW_WIN // CHUNK):
                r = row0 + j * CHUNK
                pltpu.sync_copy(acc.at[pl.ds(r, CHUNK), :], feat_v)
                pltpu.sync_copy(feat_v, sum_out.at[pl.ds(r, CHUNK), :])
            pltpu.sync_copy(deg.at[pl.ds(row0, ROW_WIN)], dstage_v)
            pltpu.sync_copy(dstage_v, deg_out.at[pl.ds(row0, ROW_WIN)])

        pl.when(c == 0)(lambda: writeback(sum_w_hbm, deg_w_hbm))
        pl.when(c == 1)(lambda: writeback(sum_l_hbm, deg_l_hbm))

    return sc_kernel(win_feat, win_dst, loss_feat, loss_dst, zeros_h)


BN_ROWS = 2000


def _readout_body(sw_ref, sl_ref, dw_ref, dl_ref,
                  g_ref, bt_ref, rm_ref, rv_ref, w_ref, b_ref, o_ref):
    dw = dw_ref[...]
    dl = dl_ref[...]
    mw = sw_ref[...] / jnp.maximum(dw, 1.0)
    ml = sl_ref[...] / jnp.maximum(dl, 1.0)
    hw = (dw > 0.0).astype(jnp.float32)
    hl = (dl > 0.0).astype(jnp.float32)
    cnt = jnp.maximum(hw + hl, 1.0)
    h = (mw * hw + ml * hl) / cnt
    hb = (h - rm_ref[...]) * lax.rsqrt(rv_ref[...] + EPS) * g_ref[...] + bt_ref[...]
    y = jnp.dot(hb, w_ref[...], preferred_element_type=jnp.float32) + b_ref[...]
    o_ref[...] = jnp.maximum(y, 0.0)


def _readout(sum_w, deg_w, sum_l, deg_l, r_gamma, r_beta, r_rm, r_rv, W3, b3):
    grid = (N_NODES // BN_ROWS,)
    row_spec = pl.BlockSpec((BN_ROWS, D), lambda i: (i, 0))
    deg_spec = pl.BlockSpec((BN_ROWS, 1), lambda i: (i, 0))
    vec_spec = pl.BlockSpec((1, D), lambda i: (0, 0))
    mat_spec = pl.BlockSpec((D, D), lambda i: (0, 0))
    return pl.pallas_call(
        _readout_body,
        grid=grid,
        in_specs=[row_spec, row_spec, deg_spec, deg_spec,
                  vec_spec, vec_spec, vec_spec, vec_spec, mat_spec, vec_spec],
        out_specs=row_spec,
        out_shape=jax.ShapeDtypeStruct((N_NODES, D), jnp.float32),
    )(sum_w, sum_l, deg_w.reshape(N_NODES, 1), deg_l.reshape(N_NODES, 1),
      r_gamma.reshape(1, D), r_beta.reshape(1, D),
      r_rm.reshape(1, D), r_rv.reshape(1, D), W3, b3.reshape(1, D))


def kernel(win_feat, loss_feat, win_dst, loss_dst,
           r_gamma, r_beta, r_rm, r_rv, W3, b3):
    sum_w, sum_l, deg_w, deg_l = _sc_segment_sums(
        win_feat, win_dst, loss_feat, loss_dst)
    return _readout(sum_w, deg_w, sum_l, deg_l,
                    r_gamma, r_beta, r_rm, r_rv, W3, b3)


# speedup vs baseline: 8.5939x; 2.0817x over previous
import functools

import jax
import jax.numpy as jnp
from jax import lax
from jax.experimental import pallas as pl
from jax.experimental.pallas import tpu as pltpu
from jax.experimental.pallas import tpu_sc as plsc

N_NODES = 10000
E = 320000
D = 128
EPS = 1e-5

NUM_CORES = 2
NUM_SUBCORES = 16

EDGES_PER_TILE = E // NUM_SUBCORES
CHUNK = 80
CHUNKS_PER_TILE = EDGES_PER_TILE // CHUNK
ROW_STRIDE = 624
ROW_WIN = 640

def _sc_segment_sums(win_feat, win_dst, loss_feat, loss_dst):
    zeros_h = jnp.zeros((N_NODES, D), dtype=jnp.float32)

    mesh = plsc.VectorSubcoreMesh(
        core_axis_name="c", subcore_axis_name="s",
        num_cores=NUM_CORES, num_subcores=NUM_SUBCORES)

    @functools.partial(
        pl.kernel,
        out_type=[
            jax.ShapeDtypeStruct((N_NODES, D), jnp.float32),
            jax.ShapeDtypeStruct((N_NODES, D), jnp.float32),
            jax.ShapeDtypeStruct((N_NODES,), jnp.float32),
            jax.ShapeDtypeStruct((N_NODES,), jnp.float32),
        ],
        mesh=mesh,
        scratch_types=[
            pltpu.VMEM_SHARED((N_NODES, D), jnp.float32),
            pltpu.VMEM_SHARED((N_NODES,), jnp.float32),
            pltpu.VMEM((CHUNK,), jnp.int32),
            pltpu.VMEM((CHUNK,), jnp.int32),
            pltpu.VMEM((CHUNK, D), jnp.float32),
            pltpu.VMEM((CHUNK, D), jnp.float32),
            pltpu.VMEM((ROW_WIN,), jnp.float32),
            pltpu.VMEM((CHUNK,), jnp.float32),
            pltpu.SemaphoreType.DMA,
            pltpu.SemaphoreType.DMA,
            pltpu.SemaphoreType.DMA,
            pltpu.SemaphoreType.DMA,
        ],
    )
    def sc_kernel(wf_hbm, wd_hbm, lf_hbm, ld_hbm, z_hbm,
                  sum_w_hbm, sum_l_hbm, deg_w_hbm, deg_l_hbm,
                  acc, deg, idx0, idx1, feat0, feat1, dstage_v, ones_v,
                  s_i0, s_i1, s_f0, s_f1):
        c = lax.axis_index("c")
        s = lax.axis_index("s")
        row0 = s * ROW_STRIDE
        base_t = s * EDGES_PER_TILE

        def _fill(i, val, ref):
            ref[pl.ds(i * 16, 16)] = jnp.full((16,), val, jnp.float32)
            return val

        lax.fori_loop(0, ROW_WIN // 16,
                      lambda i, v: _fill(i, v, dstage_v), 0.0)
        lax.fori_loop(0, CHUNK // 16,
                      lambda i, v: _fill(i, v, ones_v), 1.0)

        for j in range(ROW_WIN // CHUNK):
            r = row0 + j * CHUNK
            pltpu.sync_copy(z_hbm.at[pl.ds(r, CHUNK), :], feat0)
            pltpu.sync_copy(feat0, acc.at[pl.ds(r, CHUNK), :])
        pltpu.sync_copy(dstage_v, deg.at[pl.ds(row0, ROW_WIN)])
        plsc.subcore_barrier()

        def scatter(feat_hbm, dst_hbm):
            def issue(i, idxb, featb, semi, semf):
                base = base_t + i * CHUNK
                pltpu.async_copy(dst_hbm.at[pl.ds(base, CHUNK)], idxb, semi)
                pltpu.async_copy(feat_hbm.at[pl.ds(base, CHUNK), :], featb, semf)

            def wait(i, idxb, featb, semi, semf):
                base = base_t + i * CHUNK
                pltpu.make_async_copy(
                    dst_hbm.at[pl.ds(base, CHUNK)], idxb, semi).wait()
                pltpu.make_async_copy(
                    feat_hbm.at[pl.ds(base, CHUNK), :], featb, semf).wait()

            def scat(idxb, featb):
                pltpu.sync_copy(featb, acc.at[idxb], add=True)
                pltpu.sync_copy(ones_v, deg.at[idxb], add=True)

            half = CHUNKS_PER_TILE // 2
            issue(0, idx0, feat0, s_i0, s_f0)

            def body(k, carry):
                e = 2 * k
                issue(e + 1, idx1, feat1, s_i1, s_f1)
                wait(e, idx0, feat0, s_i0, s_f0)
                scat(idx0, feat0)
                pl.when(k < half - 1)(
                    lambda: issue(e + 2, idx0, feat0, s_i0, s_f0))
                wait(e + 1, idx1, feat1, s_i1, s_f1)
                scat(idx1, feat1)
                return carry

            lax.fori_loop(0, half, body, 0)

        pl.when(c == 0)(lambda: scatter(wf_hbm, wd_hbm))
        pl.when(c == 1)(lambda: scatter(lf_hbm, ld_hbm))
        plsc.subcore_barrier()

        def writeback(sum_out, deg_out):
            for j in range(ROW_WIN // CHUNK):
                r = row0 + j * CHUNK
                pltpu.sync_copy(acc.at[pl.ds(r, CHUNK), :], feat0)
                pltpu.sync_copy(feat0, sum_out.at[pl.ds(r, CHUNK), :])
            pltpu.sync_copy(deg.at[pl.ds(row0, ROW_WIN)], dstage_v)
            pltpu.sync_copy(dstage_v, deg_out.at[pl.ds(row0, ROW_WIN)])

        pl.when(c == 0)(lambda: writeback(sum_w_hbm, deg_w_hbm))
        pl.when(c == 1)(lambda: writeback(sum_l_hbm, deg_l_hbm))

    return sc_kernel(win_feat, win_dst, loss_feat, loss_dst, zeros_h)


BN_ROWS = 2000


def _readout_body(sw_ref, sl_ref, dw_ref, dl_ref,
                  g_ref, bt_ref, rm_ref, rv_ref, w_ref, b_ref, o_ref):
    dw = dw_ref[...]
    dl = dl_ref[...]
    mw = sw_ref[...] / jnp.maximum(dw, 1.0)
    ml = sl_ref[...] / jnp.maximum(dl, 1.0)
    hw = (dw > 0.0).astype(jnp.float32)
    hl = (dl > 0.0).astype(jnp.float32)
    cnt = jnp.maximum(hw + hl, 1.0)
    h = (mw * hw + ml * hl) / cnt
    hb = (h - rm_ref[...]) * lax.rsqrt(rv_ref[...] + EPS) * g_ref[...] + bt_ref[...]
    y = jnp.dot(hb, w_ref[...], preferred_element_type=jnp.float32) + b_ref[...]
    o_ref[...] = jnp.maximum(y, 0.0)


def _readout(sum_w, deg_w, sum_l, deg_l, r_gamma, r_beta, r_rm, r_rv, W3, b3):
    grid = (N_NODES // BN_ROWS,)
    row_spec = pl.BlockSpec((BN_ROWS, D), lambda i: (i, 0))
    deg_spec = pl.BlockSpec((BN_ROWS, 1), lambda i: (i, 0))
    vec_spec = pl.BlockSpec((1, D), lambda i: (0, 0))
    mat_spec = pl.BlockSpec((D, D), lambda i: (0, 0))
    return pl.pallas_call(
        _readout_body,
        grid=grid,
        in_specs=[row_spec, row_spec, deg_spec, deg_spec,
                  vec_spec, vec_spec, vec_spec, vec_spec, mat_spec, vec_spec],
        out_specs=row_spec,
        out_shape=jax.ShapeDtypeStruct((N_NODES, D), jnp.float32),
    )(sum_w, sum_l, deg_w.reshape(N_NODES, 1), deg_l.reshape(N_NODES, 1),
      r_gamma.reshape(1, D), r_beta.reshape(1, D),
      r_rm.reshape(1, D), r_rv.reshape(1, D), W3, b3.reshape(1, D))


def kernel(win_feat, loss_feat, win_dst, loss_dst,
           r_gamma, r_beta, r_rm, r_rv, W3, b3):
    sum_w, sum_l, deg_w, deg_l = _sc_segment_sums(
        win_feat, win_dst, loss_feat, loss_dst)
    return _readout(sum_w, deg_w, sum_l, deg_l,
                    r_gamma, r_beta, r_rm, r_rv, W3, b3)


# speedup vs baseline: 9.4299x; 1.0973x over previous
import functools

import jax
import jax.numpy as jnp
from jax import lax
from jax.experimental import pallas as pl
from jax.experimental.pallas import tpu as pltpu
from jax.experimental.pallas import tpu_sc as plsc

N_NODES = 10000
E = 320000
D = 128
EPS = 1e-5

NUM_CORES = 2
NUM_SUBCORES = 16

EDGES_PER_TILE = E // NUM_SUBCORES
CHUNK = 128
FULL_CHUNKS = EDGES_PER_TILE // CHUNK
TAIL = EDGES_PER_TILE - FULL_CHUNKS * CHUNK
ROW_STRIDE = 624
ROW_WIN = 640

def _sc_segment_sums(win_feat, win_dst, loss_feat, loss_dst):
    zeros_h = jnp.zeros((N_NODES, D), dtype=jnp.float32)

    mesh = plsc.VectorSubcoreMesh(
        core_axis_name="c", subcore_axis_name="s",
        num_cores=NUM_CORES, num_subcores=NUM_SUBCORES)

    @functools.partial(
        pl.kernel,
        out_type=[
            jax.ShapeDtypeStruct((N_NODES, D), jnp.float32),
            jax.ShapeDtypeStruct((N_NODES, D), jnp.float32),
            jax.ShapeDtypeStruct((N_NODES,), jnp.float32),
            jax.ShapeDtypeStruct((N_NODES,), jnp.float32),
        ],
        mesh=mesh,
        scratch_types=[
            pltpu.VMEM_SHARED((N_NODES, D), jnp.float32),
            pltpu.VMEM_SHARED((N_NODES,), jnp.float32),
            pltpu.VMEM((CHUNK,), jnp.int32),
            pltpu.VMEM((CHUNK,), jnp.int32),
            pltpu.VMEM((CHUNK, D), jnp.float32),
            pltpu.VMEM((CHUNK, D), jnp.float32),
            pltpu.VMEM((TAIL,), jnp.int32),
            pltpu.VMEM((TAIL, D), jnp.float32),
            pltpu.VMEM((ROW_WIN,), jnp.float32),
            pltpu.VMEM((CHUNK,), jnp.float32),
            pltpu.SemaphoreType.DMA,
            pltpu.SemaphoreType.DMA,
            pltpu.SemaphoreType.DMA,
            pltpu.SemaphoreType.DMA,
        ],
    )
    def sc_kernel(wf_hbm, wd_hbm, lf_hbm, ld_hbm, z_hbm,
                  sum_w_hbm, sum_l_hbm, deg_w_hbm, deg_l_hbm,
                  acc, deg, idx0, idx1, feat0, feat1, idxT, featT,
                  dstage_v, ones_v, s_i0, s_i1, s_f0, s_f1):
        c = lax.axis_index("c")
        s = lax.axis_index("s")
        row0 = s * ROW_STRIDE
        base_t = s * EDGES_PER_TILE

        def _fill(i, val, ref):
            ref[pl.ds(i * 16, 16)] = jnp.full((16,), val, jnp.float32)
            return val

        lax.fori_loop(0, ROW_WIN // 16,
                      lambda i, v: _fill(i, v, dstage_v), 0.0)
        lax.fori_loop(0, CHUNK // 16,
                      lambda i, v: _fill(i, v, ones_v), 1.0)

        for j in range(ROW_WIN // CHUNK):
            r = row0 + j * CHUNK
            pltpu.sync_copy(z_hbm.at[pl.ds(r, CHUNK), :], feat0)
            pltpu.sync_copy(feat0, acc.at[pl.ds(r, CHUNK), :])
        pltpu.sync_copy(dstage_v, deg.at[pl.ds(row0, ROW_WIN)])
        plsc.subcore_barrier()

        def scatter(feat_hbm, dst_hbm):
            def issue(i, idxb, featb, semi, semf):
                base = base_t + i * CHUNK
                pltpu.async_copy(dst_hbm.at[pl.ds(base, CHUNK)], idxb, semi)
                pltpu.async_copy(feat_hbm.at[pl.ds(base, CHUNK), :], featb, semf)

            def wait(i, idxb, featb, semi, semf):
                base = base_t + i * CHUNK
                pltpu.make_async_copy(
                    dst_hbm.at[pl.ds(base, CHUNK)], idxb, semi).wait()
                pltpu.make_async_copy(
                    feat_hbm.at[pl.ds(base, CHUNK), :], featb, semf).wait()

            def scat(idxb, featb):
                pltpu.sync_copy(featb, acc.at[idxb], add=True)
                pltpu.sync_copy(ones_v, deg.at[idxb], add=True)

            half = FULL_CHUNKS // 2
            issue(0, idx0, feat0, s_i0, s_f0)

            def body(k, carry):
                e = 2 * k
                issue(e + 1, idx1, feat1, s_i1, s_f1)
                wait(e, idx0, feat0, s_i0, s_f0)
                scat(idx0, feat0)
                pl.when(k < half - 1)(
                    lambda: issue(e + 2, idx0, feat0, s_i0, s_f0))
                wait(e + 1, idx1, feat1, s_i1, s_f1)
                scat(idx1, feat1)
                return carry

            lax.fori_loop(0, half, body, 0)

            tbase = base_t + FULL_CHUNKS * CHUNK
            pltpu.sync_copy(dst_hbm.at[pl.ds(tbase, TAIL)], idxT)
            pltpu.sync_copy(feat_hbm.at[pl.ds(tbase, TAIL), :], featT)
            pltpu.sync_copy(featT, acc.at[idxT], add=True)
            pltpu.sync_copy(ones_v.at[pl.ds(0, TAIL)], deg.at[idxT], add=True)

        pl.when(c == 0)(lambda: scatter(wf_hbm, wd_hbm))
        pl.when(c == 1)(lambda: scatter(lf_hbm, ld_hbm))
        plsc.subcore_barrier()

        def writeback(sum_out, deg_out):
            for j in range(ROW_WIN // CHUNK):
                r = row0 + j * CHUNK
                pltpu.sync_copy(acc.at[pl.ds(r, CHUNK), :], feat0)
                pltpu.sync_copy(feat0, sum_out.at[pl.ds(r, CHUNK), :])
            pltpu.sync_copy(deg.at[pl.ds(row0, ROW_WIN)], dstage_v)
            pltpu.sync_copy(dstage_v, deg_out.at[pl.ds(row0, ROW_WIN)])

        pl.when(c == 0)(lambda: writeback(sum_w_hbm, deg_w_hbm))
        pl.when(c == 1)(lambda: writeback(sum_l_hbm, deg_l_hbm))

    return sc_kernel(win_feat, win_dst, loss_feat, loss_dst, zeros_h)


BN_ROWS = 2000


def _readout_body(sw_ref, sl_ref, dw_ref, dl_ref,
                  g_ref, bt_ref, rm_ref, rv_ref, w_ref, b_ref, o_ref):
    dw = dw_ref[...]
    dl = dl_ref[...]
    mw = sw_ref[...] / jnp.maximum(dw, 1.0)
    ml = sl_ref[...] / jnp.maximum(dl, 1.0)
    hw = (dw > 0.0).astype(jnp.float32)
    hl = (dl > 0.0).astype(jnp.float32)
    cnt = jnp.maximum(hw + hl, 1.0)
    h = (mw * hw + ml * hl) / cnt
    hb = (h - rm_ref[...]) * lax.rsqrt(rv_ref[...] + EPS) * g_ref[...] + bt_ref[...]
    y = jnp.dot(hb, w_ref[...], preferred_element_type=jnp.float32) + b_ref[...]
    o_ref[...] = jnp.maximum(y, 0.0)


def _readout(sum_w, deg_w, sum_l, deg_l, r_gamma, r_beta, r_rm, r_rv, W3, b3):
    grid = (N_NODES // BN_ROWS,)
    row_spec = pl.BlockSpec((BN_ROWS, D), lambda i: (i, 0))
    deg_spec = pl.BlockSpec((BN_ROWS, 1), lambda i: (i, 0))
    vec_spec = pl.BlockSpec((1, D), lambda i: (0, 0))
    mat_spec = pl.BlockSpec((D, D), lambda i: (0, 0))
    return pl.pallas_call(
        _readout_body,
        grid=grid,
        in_specs=[row_spec, row_spec, deg_spec, deg_spec,
                  vec_spec, vec_spec, vec_spec, vec_spec, mat_spec, vec_spec],
        out_specs=row_spec,
        out_shape=jax.ShapeDtypeStruct((N_NODES, D), jnp.float32),
    )(sum_w, sum_l, deg_w.reshape(N_NODES, 1), deg_l.reshape(N_NODES, 1),
      r_gamma.reshape(1, D), r_beta.reshape(1, D),
      r_rm.reshape(1, D), r_rv.reshape(1, D), W3, b3.reshape(1, D))


def kernel(win_feat, loss_feat, win_dst, loss_dst,
           r_gamma, r_beta, r_rm, r_rv, W3, b3):
    sum_w, sum_l, deg_w, deg_l = _sc_segment_sums(
        win_feat, win_dst, loss_feat, loss_dst)
    return _readout(sum_w, deg_w, sum_l, deg_l,
                    r_gamma, r_beta, r_rm, r_rv, W3, b3)


# speedup vs baseline: 9.7112x; 1.0298x over previous
import functools

import jax
import jax.numpy as jnp
from jax import lax
from jax.experimental import pallas as pl
from jax.experimental.pallas import tpu as pltpu
from jax.experimental.pallas import tpu_sc as plsc

N_NODES = 10000
E = 320000
D = 128
EPS = 1e-5

NUM_CORES = 2
NUM_SUBCORES = 16

EDGES_PER_TILE = E // NUM_SUBCORES
CHUNK = 128
FULL_CHUNKS = EDGES_PER_TILE // CHUNK
TAIL = EDGES_PER_TILE - FULL_CHUNKS * CHUNK
ROW_STRIDE = 624
ROW_WIN = 640

def _sc_segment_sums(win_feat, win_dst, loss_feat, loss_dst):
    zeros_h = jnp.zeros((N_NODES, D), dtype=jnp.float32)

    mesh = plsc.VectorSubcoreMesh(
        core_axis_name="c", subcore_axis_name="s",
        num_cores=NUM_CORES, num_subcores=NUM_SUBCORES)

    @functools.partial(
        pl.kernel,
        out_type=[
            jax.ShapeDtypeStruct((N_NODES, D), jnp.float32),
            jax.ShapeDtypeStruct((N_NODES, D), jnp.float32),
            jax.ShapeDtypeStruct((N_NODES,), jnp.float32),
            jax.ShapeDtypeStruct((N_NODES,), jnp.float32),
        ],
        mesh=mesh,
        scratch_types=[
            pltpu.VMEM_SHARED((N_NODES, D), jnp.float32),
            pltpu.VMEM_SHARED((N_NODES,), jnp.float32),
            [pltpu.VMEM((CHUNK,), jnp.int32)] * 3,
            [pltpu.VMEM((CHUNK, D), jnp.float32)] * 3,
            pltpu.VMEM((TAIL,), jnp.int32),
            pltpu.VMEM((ROW_WIN,), jnp.float32),
            pltpu.VMEM((CHUNK,), jnp.float32),
            [pltpu.SemaphoreType.DMA] * 3,
            [pltpu.SemaphoreType.DMA] * 3,
        ],
    )
    def sc_kernel(wf_hbm, wd_hbm, lf_hbm, ld_hbm, z_hbm,
                  sum_w_hbm, sum_l_hbm, deg_w_hbm, deg_l_hbm,
                  acc, deg, idxb, featb, idxT,
                  dstage_v, ones_v, s_i, s_f):
        c = lax.axis_index("c")
        s = lax.axis_index("s")
        row0 = s * ROW_STRIDE
        base_t = s * EDGES_PER_TILE

        def _fill(i, val, ref):
            ref[pl.ds(i * 16, 16)] = jnp.full((16,), val, jnp.float32)
            return val

        lax.fori_loop(0, ROW_WIN // 16,
                      lambda i, v: _fill(i, v, dstage_v), 0.0)
        lax.fori_loop(0, CHUNK // 16,
                      lambda i, v: _fill(i, v, ones_v), 1.0)

        pltpu.sync_copy(z_hbm.at[pl.ds(row0, CHUNK), :], featb[0])
        for j in range(ROW_WIN // CHUNK):
            r = row0 + j * CHUNK
            pltpu.sync_copy(featb[0], acc.at[pl.ds(r, CHUNK), :])
        pltpu.sync_copy(dstage_v, deg.at[pl.ds(row0, ROW_WIN)])
        plsc.subcore_barrier()

        NBUF = 3

        def scatter(feat_hbm, dst_hbm):
            def issue(i, b):
                base = base_t + i * CHUNK
                pltpu.async_copy(dst_hbm.at[pl.ds(base, CHUNK)], idxb[b], s_i[b])
                pltpu.async_copy(feat_hbm.at[pl.ds(base, CHUNK), :], featb[b], s_f[b])

            def wait(i, b):
                base = base_t + i * CHUNK
                pltpu.make_async_copy(
                    dst_hbm.at[pl.ds(base, CHUNK)], idxb[b], s_i[b]).wait()
                pltpu.make_async_copy(
                    feat_hbm.at[pl.ds(base, CHUNK), :], featb[b], s_f[b]).wait()

            def scat(b):
                pltpu.sync_copy(featb[b], acc.at[idxb[b]], add=True)
                pltpu.sync_copy(ones_v, deg.at[idxb[b]], add=True)

            for b in range(NBUF - 1):
                issue(b, b)

            def body(k, carry):
                for jj in range(NBUF):
                    q = NBUF * k + jj
                    wait(q, jj)
                    scat(jj)
                    nq = q + NBUF - 1
                    pl.when(nq < FULL_CHUNKS)(
                        lambda jj=jj, nq=nq: issue(nq, (jj + NBUF - 1) % NBUF))
                return carry

            lax.fori_loop(0, FULL_CHUNKS // NBUF, body, 0)

            tbase = base_t + FULL_CHUNKS * CHUNK
            pltpu.sync_copy(dst_hbm.at[pl.ds(tbase, TAIL)], idxT)
            pltpu.sync_copy(feat_hbm.at[pl.ds(tbase, TAIL), :],
                            featb[0].at[pl.ds(0, TAIL), :])
            pltpu.sync_copy(featb[0].at[pl.ds(0, TAIL), :],
                            acc.at[idxT], add=True)
            pltpu.sync_copy(ones_v.at[pl.ds(0, TAIL)], deg.at[idxT], add=True)

        pl.when(c == 0)(lambda: scatter(wf_hbm, wd_hbm))
        pl.when(c == 1)(lambda: scatter(lf_hbm, ld_hbm))
        plsc.subcore_barrier()

        def writeback(sum_out, deg_out):
            for j in range(ROW_WIN // CHUNK):
                r = row0 + j * CHUNK
                pltpu.sync_copy(acc.at[pl.ds(r, CHUNK), :], featb[j % 2])
                pltpu.sync_copy(featb[j % 2], sum_out.at[pl.ds(r, CHUNK), :])
            pltpu.sync_copy(deg.at[pl.ds(row0, ROW_WIN)], dstage_v)
            pltpu.sync_copy(dstage_v, deg_out.at[pl.ds(row0, ROW_WIN)])

        pl.when(c == 0)(lambda: writeback(sum_w_hbm, deg_w_hbm))
        pl.when(c == 1)(lambda: writeback(sum_l_hbm, deg_l_hbm))

    return sc_kernel(win_feat, win_dst, loss_feat, loss_dst, zeros_h)


BN_ROWS = 2000


def _readout_body(sw_ref, sl_ref, dw_ref, dl_ref,
                  g_ref, bt_ref, rm_ref, rv_ref, w_ref, b_ref, o_ref):
    dw = dw_ref[...]
    dl = dl_ref[...]
    mw = sw_ref[...] / jnp.maximum(dw, 1.0)
    ml = sl_ref[...] / jnp.maximum(dl, 1.0)
    hw = (dw > 0.0).astype(jnp.float32)
    hl = (dl > 0.0).astype(jnp.float32)
    cnt = jnp.maximum(hw + hl, 1.0)
    h = (mw * hw + ml * hl) / cnt
    hb = (h - rm_ref[...]) * lax.rsqrt(rv_ref[...] + EPS) * g_ref[...] + bt_ref[...]
    y = jnp.dot(hb, w_ref[...], preferred_element_type=jnp.float32) + b_ref[...]
    o_ref[...] = jnp.maximum(y, 0.0)


def _readout(sum_w, deg_w, sum_l, deg_l, r_gamma, r_beta, r_rm, r_rv, W3, b3):
    grid = (N_NODES // BN_ROWS,)
    row_spec = pl.BlockSpec((BN_ROWS, D), lambda i: (i, 0))
    deg_spec = pl.BlockSpec((BN_ROWS, 1), lambda i: (i, 0))
    vec_spec = pl.BlockSpec((1, D), lambda i: (0, 0))
    mat_spec = pl.BlockSpec((D, D), lambda i: (0, 0))
    return pl.pallas_call(
        _readout_body,
        grid=grid,
        in_specs=[row_spec, row_spec, deg_spec, deg_spec,
                  vec_spec, vec_spec, vec_spec, vec_spec, mat_spec, vec_spec],
        out_specs=row_spec,
        out_shape=jax.ShapeDtypeStruct((N_NODES, D), jnp.float32),
    )(sum_w, sum_l, deg_w.reshape(N_NODES, 1), deg_l.reshape(N_NODES, 1),
      r_gamma.reshape(1, D), r_beta.reshape(1, D),
      r_rm.reshape(1, D), r_rv.reshape(1, D), W3, b3.reshape(1, D))


def kernel(win_feat, loss_feat, win_dst, loss_dst,
           r_gamma, r_beta, r_rm, r_rv, W3, b3):
    sum_w, sum_l, deg_w, deg_l = _sc_segment_sums(
        win_feat, win_dst, loss_feat, loss_dst)
    return _readout(sum_w, deg_w, sum_l, deg_l,
                    r_gamma, r_beta, r_rm, r_rv, W3, b3)


# speedup vs baseline: 10.2024x; 1.0506x over previous
import functools

import jax
import jax.numpy as jnp
from jax import lax
from jax.experimental import pallas as pl
from jax.experimental.pallas import tpu as pltpu
from jax.experimental.pallas import tpu_sc as plsc

N_NODES = 10000
E = 320000
D = 128
EPS = 1e-5

NUM_CORES = 2
NUM_SUBCORES = 16

EDGES_PER_TILE = E // NUM_SUBCORES
CHUNK = 128
FULL_CHUNKS = EDGES_PER_TILE // CHUNK
TAIL = EDGES_PER_TILE - FULL_CHUNKS * CHUNK
ROW_STRIDE = 624
ROW_WIN = 640

def _sc_segment_sums(win_feat, win_dst, loss_feat, loss_dst):
    zeros_h = jnp.zeros((N_NODES, D), dtype=jnp.float32)

    mesh = plsc.VectorSubcoreMesh(
        core_axis_name="c", subcore_axis_name="s",
        num_cores=NUM_CORES, num_subcores=NUM_SUBCORES)

    @functools.partial(
        pl.kernel,
        out_type=[
            jax.ShapeDtypeStruct((N_NODES, D), jnp.float32),
            jax.ShapeDtypeStruct((N_NODES, D), jnp.float32),
            jax.ShapeDtypeStruct((N_NODES,), jnp.float32),
            jax.ShapeDtypeStruct((N_NODES,), jnp.float32),
        ],
        mesh=mesh,
        scratch_types=[
            pltpu.VMEM_SHARED((N_NODES, D), jnp.float32),
            pltpu.VMEM_SHARED((N_NODES,), jnp.float32),
            [pltpu.VMEM((CHUNK,), jnp.int32)] * 3,
            [pltpu.VMEM((CHUNK, D), jnp.float32)] * 3,
            pltpu.VMEM((TAIL,), jnp.int32),
            pltpu.VMEM((ROW_WIN,), jnp.float32),
            pltpu.VMEM((CHUNK,), jnp.float32),
            [pltpu.SemaphoreType.DMA] * 3,
            [pltpu.SemaphoreType.DMA] * 3,
            [pltpu.SemaphoreType.DMA] * 3,
            [pltpu.SemaphoreType.DMA] * 3,
        ],
    )
    def sc_kernel(wf_hbm, wd_hbm, lf_hbm, ld_hbm, z_hbm,
                  sum_w_hbm, sum_l_hbm, deg_w_hbm, deg_l_hbm,
                  acc, deg, idxb, featb, idxT,
                  dstage_v, ones_v, s_i, s_f, s_sf, s_sd):
        c = lax.axis_index("c")
        s = lax.axis_index("s")
        row0 = s * ROW_STRIDE
        base_t = s * EDGES_PER_TILE

        def _fill(i, val, ref):
            ref[pl.ds(i * 16, 16)] = jnp.full((16,), val, jnp.float32)
            return val

        lax.fori_loop(0, ROW_WIN // 16,
                      lambda i, v: _fill(i, v, dstage_v), 0.0)
        lax.fori_loop(0, CHUNK // 16,
                      lambda i, v: _fill(i, v, ones_v), 1.0)

        pltpu.sync_copy(z_hbm.at[pl.ds(row0, CHUNK), :], featb[0])
        for j in range(ROW_WIN // CHUNK):
            r = row0 + j * CHUNK
            pltpu.sync_copy(featb[0], acc.at[pl.ds(r, CHUNK), :])
        pltpu.sync_copy(dstage_v, deg.at[pl.ds(row0, ROW_WIN)])
        plsc.subcore_barrier()

        NBUF = 3

        def scatter(feat_hbm, dst_hbm):
            def issue(i, b):
                base = base_t + i * CHUNK
                pltpu.async_copy(dst_hbm.at[pl.ds(base, CHUNK)], idxb[b], s_i[b])
                pltpu.async_copy(feat_hbm.at[pl.ds(base, CHUNK), :], featb[b], s_f[b])

            def wait(i, b):
                base = base_t + i * CHUNK
                pltpu.make_async_copy(
                    dst_hbm.at[pl.ds(base, CHUNK)], idxb[b], s_i[b]).wait()
                pltpu.make_async_copy(
                    feat_hbm.at[pl.ds(base, CHUNK), :], featb[b], s_f[b]).wait()

            def scat_issue(b):
                pltpu.async_copy(featb[b], acc.at[idxb[b]], s_sf[b], add=True)
                pltpu.async_copy(ones_v, deg.at[idxb[b]], s_sd[b], add=True)

            def scat_wait(b):
                pltpu.make_async_copy(featb[b], acc.at[idxb[b]], s_sf[b]).wait()
                pltpu.make_async_copy(ones_v, deg.at[idxb[b]], s_sd[b]).wait()

            issue(0, 0)
            issue(1, 1)
            wait(0, 0)
            scat_issue(0)
            issue(2, 2)
            wait(1, 1)
            scat_issue(1)
            scat_wait(0)
            issue(3, 0)
            wait(2, 2)
            scat_issue(2)
            scat_wait(1)
            issue(4, 1)

            def body(k, carry):
                for jj in range(NBUF):
                    q = NBUF * k + jj
                    wait(q, jj)
                    scat_issue(jj)
                    b2 = (jj + NBUF - 1) % NBUF
                    nq = q + NBUF - 1

                    def refill(b2=b2, nq=nq):
                        scat_wait(b2)
                        issue(nq, b2)

                    pl.when(nq < FULL_CHUNKS)(refill)
                return carry

            lax.fori_loop(1, FULL_CHUNKS // NBUF, body, 0)
            for b in range(NBUF):
                scat_wait(b)

            tbase = base_t + FULL_CHUNKS * CHUNK
            pltpu.sync_copy(dst_hbm.at[pl.ds(tbase, TAIL)], idxT)
            pltpu.sync_copy(feat_hbm.at[pl.ds(tbase, TAIL), :],
                            featb[0].at[pl.ds(0, TAIL), :])
            pltpu.sync_copy(featb[0].at[pl.ds(0, TAIL), :],
                            acc.at[idxT], add=True)
            pltpu.sync_copy(ones_v.at[pl.ds(0, TAIL)], deg.at[idxT], add=True)

        pl.when(c == 0)(lambda: scatter(wf_hbm, wd_hbm))
        pl.when(c == 1)(lambda: scatter(lf_hbm, ld_hbm))
        plsc.subcore_barrier()

        def writeback(sum_out, deg_out):
            for j in range(ROW_WIN // CHUNK):
                r = row0 + j * CHUNK
                pltpu.sync_copy(acc.at[pl.ds(r, CHUNK), :], featb[j % 2])
                pltpu.sync_copy(featb[j % 2], sum_out.at[pl.ds(r, CHUNK), :])
            pltpu.sync_copy(deg.at[pl.ds(row0, ROW_WIN)], dstage_v)
            pltpu.sync_copy(dstage_v, deg_out.at[pl.ds(row0, ROW_WIN)])

        pl.when(c == 0)(lambda: writeback(sum_w_hbm, deg_w_hbm))
        pl.when(c == 1)(lambda: writeback(sum_l_hbm, deg_l_hbm))

    return sc_kernel(win_feat, win_dst, loss_feat, loss_dst, zeros_h)


BN_ROWS = 2000


def _readout_body(sw_ref, sl_ref, dw_ref, dl_ref,
                  g_ref, bt_ref, rm_ref, rv_ref, w_ref, b_ref, o_ref):
    dw = dw_ref[...]
    dl = dl_ref[...]
    mw = sw_ref[...] / jnp.maximum(dw, 1.0)
    ml = sl_ref[...] / jnp.maximum(dl, 1.0)
    hw = (dw > 0.0).astype(jnp.float32)
    hl = (dl > 0.0).astype(jnp.float32)
    cnt = jnp.maximum(hw + hl, 1.0)
    h = (mw * hw + ml * hl) / cnt
    hb = (h - rm_ref[...]) * lax.rsqrt(rv_ref[...] + EPS) * g_ref[...] + bt_ref[...]
    y = jnp.dot(hb, w_ref[...], preferred_element_type=jnp.float32) + b_ref[...]
    o_ref[...] = jnp.maximum(y, 0.0)


def _readout(sum_w, deg_w, sum_l, deg_l, r_gamma, r_beta, r_rm, r_rv, W3, b3):
    grid = (N_NODES // BN_ROWS,)
    row_spec = pl.BlockSpec((BN_ROWS, D), lambda i: (i, 0))
    deg_spec = pl.BlockSpec((BN_ROWS, 1), lambda i: (i, 0))
    vec_spec = pl.BlockSpec((1, D), lambda i: (0, 0))
    mat_spec = pl.BlockSpec((D, D), lambda i: (0, 0))
    return pl.pallas_call(
        _readout_body,
        grid=grid,
        in_specs=[row_spec, row_spec, deg_spec, deg_spec,
                  vec_spec, vec_spec, vec_spec, vec_spec, mat_spec, vec_spec],
        out_specs=row_spec,
        out_shape=jax.ShapeDtypeStruct((N_NODES, D), jnp.float32),
    )(sum_w, sum_l, deg_w.reshape(N_NODES, 1), deg_l.reshape(N_NODES, 1),
      r_gamma.reshape(1, D), r_beta.reshape(1, D),
      r_rm.reshape(1, D), r_rv.reshape(1, D), W3, b3.reshape(1, D))


def kernel(win_feat, loss_feat, win_dst, loss_dst,
           r_gamma, r_beta, r_rm, r_rv, W3, b3):
    sum_w, sum_l, deg_w, deg_l = _sc_segment_sums(
        win_feat, win_dst, loss_feat, loss_dst)
    return _readout(sum_w, deg_w, sum_l, deg_l,
                    r_gamma, r_beta, r_rm, r_rv, W3, b3)


# speedup vs baseline: 10.3272x; 1.0122x over previous
import functools

import jax
import jax.numpy as jnp
from jax import lax
from jax.experimental import pallas as pl
from jax.experimental.pallas import tpu as pltpu
from jax.experimental.pallas import tpu_sc as plsc

N_NODES = 10000
E = 320000
D = 128
EPS = 1e-5

NUM_CORES = 2
NUM_SUBCORES = 16

EDGES_PER_TILE = E // NUM_SUBCORES
CHUNK = 128
FULL_CHUNKS = EDGES_PER_TILE // CHUNK
TAIL = EDGES_PER_TILE - FULL_CHUNKS * CHUNK
ROW_STRIDE = 624
ROW_WIN = 640

def _sc_segment_sums(win_feat, win_dst, loss_feat, loss_dst):
    zeros_h = jnp.zeros((N_NODES, D), dtype=jnp.float32)

    mesh = plsc.VectorSubcoreMesh(
        core_axis_name="c", subcore_axis_name="s",
        num_cores=NUM_CORES, num_subcores=NUM_SUBCORES)

    @functools.partial(
        pl.kernel,
        out_type=[
            jax.ShapeDtypeStruct((N_NODES, D), jnp.float32),
            jax.ShapeDtypeStruct((N_NODES, D), jnp.float32),
            jax.ShapeDtypeStruct((N_NODES,), jnp.float32),
            jax.ShapeDtypeStruct((N_NODES,), jnp.float32),
        ],
        mesh=mesh,
        scratch_types=[
            pltpu.VMEM_SHARED((N_NODES, D), jnp.float32),
            pltpu.VMEM_SHARED((N_NODES,), jnp.float32),
            [pltpu.VMEM((CHUNK,), jnp.int32)] * 3,
            [pltpu.VMEM((CHUNK, D), jnp.float32)] * 3,
            pltpu.VMEM((TAIL,), jnp.int32),
            pltpu.VMEM((ROW_WIN,), jnp.float32),
            pltpu.VMEM((CHUNK,), jnp.float32),
            [pltpu.SemaphoreType.DMA] * 3,
            [pltpu.SemaphoreType.DMA] * 3,
            [pltpu.SemaphoreType.DMA] * 3,
            [pltpu.SemaphoreType.DMA] * 3,
        ],
    )
    def sc_kernel(wf_hbm, wd_hbm, lf_hbm, ld_hbm, z_hbm,
                  sum_w_hbm, sum_l_hbm, deg_w_hbm, deg_l_hbm,
                  acc, deg, idxb, featb, idxT,
                  dstage_v, ones_v, s_i, s_f, s_sf, s_sd):
        c = lax.axis_index("c")
        s = lax.axis_index("s")
        row0 = s * ROW_STRIDE
        base_t = s * EDGES_PER_TILE

        def _fill(i, val, ref):
            ref[pl.ds(i * 16, 16)] = jnp.full((16,), val, jnp.float32)
            return val

        lax.fori_loop(0, ROW_WIN // 16,
                      lambda i, v: _fill(i, v, dstage_v), 0.0)
        lax.fori_loop(0, CHUNK // 16,
                      lambda i, v: _fill(i, v, ones_v), 1.0)

        NBUF = 3

        def issue_from(feat_hbm, dst_hbm, i, b):
            base = base_t + i * CHUNK
            pltpu.async_copy(dst_hbm.at[pl.ds(base, CHUNK)], idxb[b], s_i[b])
            pltpu.async_copy(feat_hbm.at[pl.ds(base, CHUNK), :], featb[b], s_f[b])

        pl.when(c == 0)(lambda: issue_from(wf_hbm, wd_hbm, 0, 0))
        pl.when(c == 0)(lambda: issue_from(wf_hbm, wd_hbm, 1, 1))
        pl.when(c == 1)(lambda: issue_from(lf_hbm, ld_hbm, 0, 0))
        pl.when(c == 1)(lambda: issue_from(lf_hbm, ld_hbm, 1, 1))

        pltpu.sync_copy(z_hbm.at[pl.ds(row0, CHUNK), :], featb[2])
        for j in range(ROW_WIN // CHUNK):
            r = row0 + j * CHUNK
            pltpu.sync_copy(featb[2], acc.at[pl.ds(r, CHUNK), :])
        pltpu.sync_copy(dstage_v, deg.at[pl.ds(row0, ROW_WIN)])
        plsc.subcore_barrier()

        def scatter(feat_hbm, dst_hbm):
            def issue(i, b):
                issue_from(feat_hbm, dst_hbm, i, b)

            def wait(i, b):
                base = base_t + i * CHUNK
                pltpu.make_async_copy(
                    dst_hbm.at[pl.ds(base, CHUNK)], idxb[b], s_i[b]).wait()
                pltpu.make_async_copy(
                    feat_hbm.at[pl.ds(base, CHUNK), :], featb[b], s_f[b]).wait()

            def scat_issue(b):
                pltpu.async_copy(featb[b], acc.at[idxb[b]], s_sf[b], add=True)
                pltpu.async_copy(ones_v, deg.at[idxb[b]], s_sd[b], add=True)

            def scat_wait(b):
                pltpu.make_async_copy(featb[b], acc.at[idxb[b]], s_sf[b]).wait()
                pltpu.make_async_copy(ones_v, deg.at[idxb[b]], s_sd[b]).wait()

            wait(0, 0)
            scat_issue(0)
            issue(2, 2)
            wait(1, 1)
            scat_issue(1)
            scat_wait(0)
            issue(3, 0)
            wait(2, 2)
            scat_issue(2)
            scat_wait(1)
            issue(4, 1)

            def body(k, carry):
                for jj in range(NBUF):
                    q = NBUF * k + jj
                    wait(q, jj)
                    scat_issue(jj)
                    b2 = (jj + NBUF - 1) % NBUF
                    nq = q + NBUF - 1

                    def refill(b2=b2, nq=nq):
                        scat_wait(b2)
                        issue(nq, b2)

                    pl.when(nq < FULL_CHUNKS)(refill)
                return carry

            lax.fori_loop(1, FULL_CHUNKS // NBUF, body, 0)
            for b in range(NBUF):
                scat_wait(b)

            tbase = base_t + FULL_CHUNKS * CHUNK
            pltpu.sync_copy(dst_hbm.at[pl.ds(tbase, TAIL)], idxT)
            pltpu.sync_copy(feat_hbm.at[pl.ds(tbase, TAIL), :],
                            featb[0].at[pl.ds(0, TAIL), :])
            pltpu.sync_copy(featb[0].at[pl.ds(0, TAIL), :],
                            acc.at[idxT], add=True)
            pltpu.sync_copy(ones_v.at[pl.ds(0, TAIL)], deg.at[idxT], add=True)

        pl.when(c == 0)(lambda: scatter(wf_hbm, wd_hbm))
        pl.when(c == 1)(lambda: scatter(lf_hbm, ld_hbm))
        plsc.subcore_barrier()

        NWB = ROW_WIN // CHUNK

        def writeback(sum_out, deg_out):
            def rd(j, b):
                pltpu.async_copy(
                    acc.at[pl.ds(row0 + j * CHUNK, CHUNK), :], featb[b], s_f[b])

            def rdw(j, b):
                pltpu.make_async_copy(
                    acc.at[pl.ds(row0 + j * CHUNK, CHUNK), :], featb[b],
                    s_f[b]).wait()

            def wr(j, b):
                pltpu.async_copy(
                    featb[b], sum_out.at[pl.ds(row0 + j * CHUNK, CHUNK), :],
                    s_sf[b])

            def wrw(j, b):
                pltpu.make_async_copy(
                    featb[b], sum_out.at[pl.ds(row0 + j * CHUNK, CHUNK), :],
                    s_sf[b]).wait()

            rd(0, 0)
            for j in range(NWB):
                b = j % 2
                rdw(j, b)
                if j + 1 < NWB:
                    if j >= 1:
                        wrw(j - 1, (j + 1) % 2)
                    rd(j + 1, (j + 1) % 2)
                wr(j, b)
            pltpu.sync_copy(deg.at[pl.ds(row0, ROW_WIN)], dstage_v)
            pltpu.sync_copy(dstage_v, deg_out.at[pl.ds(row0, ROW_WIN)])
            wrw(NWB - 2, (NWB - 2) % 2)
            wrw(NWB - 1, (NWB - 1) % 2)

        pl.when(c == 0)(lambda: writeback(sum_w_hbm, deg_w_hbm))
        pl.when(c == 1)(lambda: writeback(sum_l_hbm, deg_l_hbm))

    return sc_kernel(win_feat, win_dst, loss_feat, loss_dst, zeros_h)


BN_ROWS = 2000


def _readout_body(sw_ref, sl_ref, dw_ref, dl_ref,
                  g_ref, bt_ref, rm_ref, rv_ref, w_ref, b_ref, o_ref):
    dw = dw_ref[...]
    dl = dl_ref[...]
    mw = sw_ref[...] / jnp.maximum(dw, 1.0)
    ml = sl_ref[...] / jnp.maximum(dl, 1.0)
    hw = (dw > 0.0).astype(jnp.float32)
    hl = (dl > 0.0).astype(jnp.float32)
    cnt = jnp.maximum(hw + hl, 1.0)
    h = (mw * hw + ml * hl) / cnt
    hb = (h - rm_ref[...]) * lax.rsqrt(rv_ref[...] + EPS) * g_ref[...] + bt_ref[...]
    y = jnp.dot(hb, w_ref[...], preferred_element_type=jnp.float32) + b_ref[...]
    o_ref[...] = jnp.maximum(y, 0.0)


def _readout(sum_w, deg_w, sum_l, deg_l, r_gamma, r_beta, r_rm, r_rv, W3, b3):
    grid = (N_NODES // BN_ROWS,)
    row_spec = pl.BlockSpec((BN_ROWS, D), lambda i: (i, 0))
    deg_spec = pl.BlockSpec((BN_ROWS, 1), lambda i: (i, 0))
    vec_spec = pl.BlockSpec((1, D), lambda i: (0, 0))
    mat_spec = pl.BlockSpec((D, D), lambda i: (0, 0))
    return pl.pallas_call(
        _readout_body,
        grid=grid,
        in_specs=[row_spec, row_spec, deg_spec, deg_spec,
                  vec_spec, vec_spec, vec_spec, vec_spec, mat_spec, vec_spec],
        out_specs=row_spec,
        out_shape=jax.ShapeDtypeStruct((N_NODES, D), jnp.float32),
    )(sum_w, sum_l, deg_w.reshape(N_NODES, 1), deg_l.reshape(N_NODES, 1),
      r_gamma.reshape(1, D), r_beta.reshape(1, D),
      r_rm.reshape(1, D), r_rv.reshape(1, D), W3, b3.reshape(1, D))


def kernel(win_feat, loss_feat, win_dst, loss_dst,
           r_gamma, r_beta, r_rm, r_rv, W3, b3):
    sum_w, sum_l, deg_w, deg_l = _sc_segment_sums(
        win_feat, win_dst, loss_feat, loss_dst)
    return _readout(sum_w, deg_w, sum_l, deg_l,
                    r_gamma, r_beta, r_rm, r_rv, W3, b3)
